# baseline (speedup 1.0000x reference)
import functools

import jax
import jax.numpy as jnp
from jax.experimental import pallas as pl
from jax.experimental.pallas import tpu as pltpu

IN_FEATURES = 4096
OUT_FEATURES = 4096
NUM_FRAGMENTS = 32
FRAGMENT_SIZE = IN_FEATURES // NUM_FRAGMENTS
COMPRESSED = 512
BATCH = 64

TILE_K = 256
GRID_K = IN_FEATURES // TILE_K
WN_COPY_STEP = GRID_K // 2

_CONTRACT_LAST = (((1,), (1,)), ((), ()))


def _frag_mask(k):
    col_frag = (k * TILE_K + jax.lax.broadcasted_iota(
        jnp.int32, (NUM_FRAGMENTS, TILE_K), 1)) // FRAGMENT_SIZE
    frag = jax.lax.broadcasted_iota(jnp.int32, (NUM_FRAGMENTS, TILE_K), 0)
    return (col_frag == frag).astype(jnp.bfloat16)


def _fused_kernel(x_ref, sw_ref, wc_ref, we_ref, wn_hbm_ref,
                  out_ref, probs_ref, comp_ref, wn_ref, wn_sem):
    k = pl.program_id(0)
    wn_copy = pltpu.make_async_copy(wn_hbm_ref, wn_ref, wn_sem)

    @pl.when(k == WN_COPY_STEP)
    def _start_wn():
        wn_copy.start()

    @pl.when(k == 0)
    def _prologue():
        xs = (x_ref[...] * sw_ref[...]).astype(jnp.bfloat16)
        masks = jnp.concatenate(
            [_frag_mask(i) for i in range(GRID_K)], axis=1)
        scores = jax.lax.dot_general(
            xs, masks, _CONTRACT_LAST, preferred_element_type=jnp.float32)
        m = jnp.max(scores, axis=1, keepdims=True)
        ex = jnp.exp(scores - m)
        probs_ref[...] = ex / jnp.sum(ex, axis=1, keepdims=True)

    xk = x_ref[:, pl.ds(k * TILE_K, TILE_K)]
    pe = jnp.dot(probs_ref[...].astype(jnp.bfloat16), _frag_mask(k),
                 preferred_element_type=jnp.float32)
    wxk = xk * pe
    expert = jnp.dot(wxk.astype(jnp.bfloat16),
                     we_ref[...].astype(jnp.bfloat16),
                     preferred_element_type=jnp.float32)
    cpart = jax.lax.dot_general((xk - wxk).astype(jnp.bfloat16),
                                wc_ref[...].astype(jnp.bfloat16),
                                _CONTRACT_LAST,
                                preferred_element_type=jnp.float32)

    @pl.when(k == 0)
    def _init():
        out_ref[...] = expert
        comp_ref[...] = cpart

    @pl.when(k > 0)
    def _accum():
        out_ref[...] += expert
        comp_ref[...] += cpart

    @pl.when(k == GRID_K - 1)
    def _epilogue():
        wn_copy.wait()
        out_ref[...] += jax.lax.dot_general(
            comp_ref[...].astype(jnp.bfloat16),
            wn_ref[...].astype(jnp.bfloat16),
            _CONTRACT_LAST, preferred_element_type=jnp.float32)


@functools.partial(jax.jit, static_argnames=())
def kernel(x, selector_weights, expert_weights, compressor_w, compressed_net_w):
    sw_row = selector_weights.reshape(1, IN_FEATURES)
    we = expert_weights.reshape(IN_FEATURES, OUT_FEATURES)

    return pl.pallas_call(
        _fused_kernel,
        grid=(GRID_K,),
        in_specs=[
            pl.BlockSpec((BATCH, IN_FEATURES), lambda k: (0, 0)),
            pl.BlockSpec((1, IN_FEATURES), lambda k: (0, 0)),
            pl.BlockSpec((COMPRESSED, TILE_K), lambda k: (0, k)),
            pl.BlockSpec((TILE_K, OUT_FEATURES), lambda k: (k, 0)),
            pl.BlockSpec(memory_space=pl.ANY),
        ],
        out_specs=pl.BlockSpec((BATCH, OUT_FEATURES), lambda k: (0, 0)),
        out_shape=jax.ShapeDtypeStruct((BATCH, OUT_FEATURES), x.dtype),
        scratch_shapes=[
            pltpu.VMEM((BATCH, NUM_FRAGMENTS), jnp.float32),
            pltpu.VMEM((BATCH, COMPRESSED), jnp.float32),
            pltpu.VMEM((OUT_FEATURES, COMPRESSED), jnp.float32),
            pltpu.SemaphoreType.DMA,
        ],
    )(x, sw_row, compressor_w, we, compressed_net_w)


# speedup vs baseline: 1.0802x; 1.0802x over previous
import functools

import jax
import jax.numpy as jnp
from jax.experimental import pallas as pl
from jax.experimental.pallas import tpu as pltpu

IN_FEATURES = 4096
OUT_FEATURES = 4096
NUM_FRAGMENTS = 32
FRAGMENT_SIZE = IN_FEATURES // NUM_FRAGMENTS
COMPRESSED = 512
BATCH = 64

TILE_K = 512
GRID_K = IN_FEATURES // TILE_K
WN_COPY_STEP = GRID_K // 2

_CONTRACT_LAST = (((1,), (1,)), ((), ()))


def _frag_mask(k):
    col_frag = (k * TILE_K + jax.lax.broadcasted_iota(
        jnp.int32, (NUM_FRAGMENTS, TILE_K), 1)) // FRAGMENT_SIZE
    frag = jax.lax.broadcasted_iota(jnp.int32, (NUM_FRAGMENTS, TILE_K), 0)
    return (col_frag == frag).astype(jnp.bfloat16)


def _fused_kernel(x_ref, sw_ref, wc_ref, we_ref, wn_hbm_ref,
                  out_ref, probs_ref, comp_ref, wn_ref, wn_sem):
    k = pl.program_id(0)
    wn_copy = pltpu.make_async_copy(wn_hbm_ref, wn_ref, wn_sem)

    @pl.when(k == WN_COPY_STEP)
    def _start_wn():
        wn_copy.start()

    @pl.when(k == 0)
    def _prologue():
        xs = (x_ref[...] * sw_ref[...]).astype(jnp.bfloat16)
        masks = jnp.concatenate(
            [_frag_mask(i) for i in range(GRID_K)], axis=1)
        scores = jax.lax.dot_general(
            xs, masks, _CONTRACT_LAST, preferred_element_type=jnp.float32)
        m = jnp.max(scores, axis=1, keepdims=True)
        ex = jnp.exp(scores - m)
        probs_ref[...] = ex / jnp.sum(ex, axis=1, keepdims=True)

    xk = x_ref[:, pl.ds(k * TILE_K, TILE_K)]
    pe = jnp.dot(probs_ref[...].astype(jnp.bfloat16), _frag_mask(k),
                 preferred_element_type=jnp.float32)
    wxk = xk * pe
    expert = jnp.dot(wxk.astype(jnp.bfloat16),
                     we_ref[...].astype(jnp.bfloat16),
                     preferred_element_type=jnp.float32)
    cpart = jax.lax.dot_general((xk - wxk).astype(jnp.bfloat16),
                                wc_ref[...].astype(jnp.bfloat16),
                                _CONTRACT_LAST,
                                preferred_element_type=jnp.float32)

    @pl.when(k == 0)
    def _init():
        out_ref[...] = expert
        comp_ref[...] = cpart

    @pl.when(k > 0)
    def _accum():
        out_ref[...] += expert
        comp_ref[...] += cpart

    @pl.when(k == GRID_K - 1)
    def _epilogue():
        wn_copy.wait()
        out_ref[...] += jax.lax.dot_general(
            comp_ref[...].astype(jnp.bfloat16),
            wn_ref[...].astype(jnp.bfloat16),
            _CONTRACT_LAST, preferred_element_type=jnp.float32)


@functools.partial(jax.jit, static_argnames=())
def kernel(x, selector_weights, expert_weights, compressor_w, compressed_net_w):
    sw_row = selector_weights.reshape(1, IN_FEATURES)
    we = expert_weights.reshape(IN_FEATURES, OUT_FEATURES)

    return pl.pallas_call(
        _fused_kernel,
        grid=(GRID_K,),
        in_specs=[
            pl.BlockSpec((BATCH, IN_FEATURES), lambda k: (0, 0)),
            pl.BlockSpec((1, IN_FEATURES), lambda k: (0, 0)),
            pl.BlockSpec((COMPRESSED, TILE_K), lambda k: (0, k)),
            pl.BlockSpec((TILE_K, OUT_FEATURES), lambda k: (k, 0)),
            pl.BlockSpec(memory_space=pl.ANY),
        ],
        out_specs=pl.BlockSpec((BATCH, OUT_FEATURES), lambda k: (0, 0)),
        out_shape=jax.ShapeDtypeStruct((BATCH, OUT_FEATURES), x.dtype),
        scratch_shapes=[
            pltpu.VMEM((BATCH, NUM_FRAGMENTS), jnp.float32),
            pltpu.VMEM((BATCH, COMPRESSED), jnp.float32),
            pltpu.VMEM((OUT_FEATURES, COMPRESSED), jnp.float32),
            pltpu.SemaphoreType.DMA,
        ],
    )(x, sw_row, compressor_w, we, compressed_net_w)


# speedup vs baseline: 1.1129x; 1.0302x over previous
import functools

import jax
import jax.numpy as jnp
from jax.experimental import pallas as pl
from jax.experimental.pallas import tpu as pltpu

IN_FEATURES = 4096
OUT_FEATURES = 4096
NUM_FRAGMENTS = 32
FRAGMENT_SIZE = IN_FEATURES // NUM_FRAGMENTS
COMPRESSED = 512
BATCH = 64

TILE_K = 512
GRID_K = IN_FEATURES // TILE_K

_CONTRACT_LAST = (((1,), (1,)), ((), ()))


def _frag_mask():
    col_frag = jax.lax.broadcasted_iota(
        jnp.int32, (NUM_FRAGMENTS, IN_FEATURES), 1) // FRAGMENT_SIZE
    frag = jax.lax.broadcasted_iota(
        jnp.int32, (NUM_FRAGMENTS, IN_FEATURES), 0)
    return (col_frag == frag).astype(jnp.bfloat16)


def _fused_kernel(x_ref, sw_ref, we_ref, wc_hbm_ref, wn_hbm_ref, out_ref,
                  wx_ref, comp_ref, wc_ref, wn_ref, wc_sem, wn_sem):
    k = pl.program_id(0)
    wc_copy = pltpu.make_async_copy(wc_hbm_ref, wc_ref, wc_sem)
    wn_copy = pltpu.make_async_copy(wn_hbm_ref, wn_ref, wn_sem)

    @pl.when(k == 0)
    def _prologue():
        wc_copy.start()
        wn_copy.start()
        xv = x_ref[...]
        masks = _frag_mask()
        xs = (xv * sw_ref[...]).astype(jnp.bfloat16)
        scores = jax.lax.dot_general(
            xs, masks, _CONTRACT_LAST, preferred_element_type=jnp.float32)
        m = jnp.max(scores, axis=1, keepdims=True)
        ex = jnp.exp(scores - m)
        probs = (ex / jnp.sum(ex, axis=1, keepdims=True)).astype(jnp.bfloat16)
        pe = jnp.dot(probs, masks, preferred_element_type=jnp.float32)
        wx_ref[...] = xv * pe

    @pl.when(k == 1)
    def _compress():
        wc_copy.wait()
        masked = (x_ref[...] - wx_ref[...]).astype(jnp.bfloat16)
        comp_ref[...] = jax.lax.dot_general(
            masked, wc_ref[...].astype(jnp.bfloat16), _CONTRACT_LAST,
            preferred_element_type=jnp.float32)

    wxk = wx_ref[:, pl.ds(k * TILE_K, TILE_K)].astype(jnp.bfloat16)
    expert = jnp.dot(wxk, we_ref[...].astype(jnp.bfloat16),
                     preferred_element_type=jnp.float32)

    @pl.when(k == 0)
    def _init():
        out_ref[...] = expert

    @pl.when(k > 0)
    def _accum():
        out_ref[...] += expert

    @pl.when(k == GRID_K - 1)
    def _epilogue():
        wn_copy.wait()
        out_ref[...] += jax.lax.dot_general(
            comp_ref[...].astype(jnp.bfloat16),
            wn_ref[...].astype(jnp.bfloat16),
            _CONTRACT_LAST, preferred_element_type=jnp.float32)


@functools.partial(jax.jit, static_argnames=())
def kernel(x, selector_weights, expert_weights, compressor_w, compressed_net_w):
    sw_row = selector_weights.reshape(1, IN_FEATURES)
    we = expert_weights.reshape(IN_FEATURES, OUT_FEATURES)

    return pl.pallas_call(
        _fused_kernel,
        grid=(GRID_K,),
        in_specs=[
            pl.BlockSpec((BATCH, IN_FEATURES), lambda k: (0, 0)),
            pl.BlockSpec((1, IN_FEATURES), lambda k: (0, 0)),
            pl.BlockSpec((TILE_K, OUT_FEATURES), lambda k: (k, 0)),
            pl.BlockSpec(memory_space=pl.ANY),
            pl.BlockSpec(memory_space=pl.ANY),
        ],
        out_specs=pl.BlockSpec((BATCH, OUT_FEATURES), lambda k: (0, 0)),
        out_shape=jax.ShapeDtypeStruct((BATCH, OUT_FEATURES), x.dtype),
        scratch_shapes=[
            pltpu.VMEM((BATCH, IN_FEATURES), jnp.float32),
            pltpu.VMEM((BATCH, COMPRESSED), jnp.float32),
            pltpu.VMEM((COMPRESSED, IN_FEATURES), jnp.float32),
            pltpu.VMEM((OUT_FEATURES, COMPRESSED), jnp.float32),
            pltpu.SemaphoreType.DMA,
            pltpu.SemaphoreType.DMA,
        ],
    )(x, sw_row, we, compressor_w, compressed_net_w)


# speedup vs baseline: 1.1405x; 1.0248x over previous
import functools

import jax
import jax.numpy as jnp
from jax.experimental import pallas as pl
from jax.experimental.pallas import tpu as pltpu

IN_FEATURES = 4096
OUT_FEATURES = 4096
NUM_FRAGMENTS = 32
FRAGMENT_SIZE = IN_FEATURES // NUM_FRAGMENTS
COMPRESSED = 512
BATCH = 64

TILE_K = 512
GRID_K = IN_FEATURES // TILE_K

_CONTRACT_LAST = (((1,), (1,)), ((), ()))


def _frag_mask():
    col_frag = jax.lax.broadcasted_iota(
        jnp.int32, (NUM_FRAGMENTS, IN_FEATURES), 1) // FRAGMENT_SIZE
    frag = jax.lax.broadcasted_iota(
        jnp.int32, (NUM_FRAGMENTS, IN_FEATURES), 0)
    return (col_frag == frag).astype(jnp.bfloat16)


def _fused_kernel(x_ref, sw_ref, we_ref, wc_hbm_ref, wn_hbm_ref, out_ref,
                  wx_ref, comp_ref, wc_ref, wn_ref, wc_sem, wn_sem):
    k = pl.program_id(0)
    wc_copy = pltpu.make_async_copy(wc_hbm_ref, wc_ref, wc_sem)
    wn_copy = pltpu.make_async_copy(wn_hbm_ref, wn_ref, wn_sem)

    @pl.when(k == 0)
    def _prologue():
        wc_copy.start()
        wn_copy.start()
        xv = x_ref[...]
        masks = _frag_mask()
        xs = (xv * sw_ref[...]).astype(jnp.bfloat16)
        scores = jax.lax.dot_general(
            xs, masks, _CONTRACT_LAST, preferred_element_type=jnp.float32)
        m = jnp.max(scores, axis=1, keepdims=True)
        ex = jnp.exp(scores - m)
        probs = (ex / jnp.sum(ex, axis=1, keepdims=True)).astype(jnp.bfloat16)
        pe = jnp.dot(probs, masks, preferred_element_type=jnp.float32)
        wx_ref[...] = xv * pe

    @pl.when(k == 1)
    def _compress():
        wc_copy.wait()
        masked = (x_ref[...] - wx_ref[...]).astype(jnp.bfloat16)
        comp_ref[...] = jax.lax.dot_general(
            masked, wc_ref[...].astype(jnp.bfloat16), _CONTRACT_LAST,
            preferred_element_type=jnp.float32)

    wxk = wx_ref[:, pl.ds(k * TILE_K, TILE_K)].astype(jnp.bfloat16)
    expert = jnp.dot(wxk, we_ref[...].astype(jnp.bfloat16),
                     preferred_element_type=jnp.float32)

    @pl.when(k == 0)
    def _init():
        out_ref[...] = expert

    @pl.when(k > 0)
    def _accum():
        out_ref[...] += expert

    half = OUT_FEATURES // 2
    cb = comp_ref[...].astype(jnp.bfloat16)

    @pl.when(k == GRID_K - 2)
    def _epilogue_lo():
        wn_copy.wait()
        out_ref[:, :half] += jax.lax.dot_general(
            cb, wn_ref[:half, :].astype(jnp.bfloat16),
            _CONTRACT_LAST, preferred_element_type=jnp.float32)

    @pl.when(k == GRID_K - 1)
    def _epilogue_hi():
        out_ref[:, half:] += jax.lax.dot_general(
            cb, wn_ref[half:, :].astype(jnp.bfloat16),
            _CONTRACT_LAST, preferred_element_type=jnp.float32)


@functools.partial(jax.jit, static_argnames=())
def kernel(x, selector_weights, expert_weights, compressor_w, compressed_net_w):
    sw_row = selector_weights.reshape(1, IN_FEATURES)
    we = expert_weights.reshape(IN_FEATURES, OUT_FEATURES)

    return pl.pallas_call(
        _fused_kernel,
        grid=(GRID_K,),
        in_specs=[
            pl.BlockSpec((BATCH, IN_FEATURES), lambda k: (0, 0)),
            pl.BlockSpec((1, IN_FEATURES), lambda k: (0, 0)),
            pl.BlockSpec((TILE_K, OUT_FEATURES), lambda k: (k, 0)),
            pl.BlockSpec(memory_space=pl.ANY),
            pl.BlockSpec(memory_space=pl.ANY),
        ],
        out_specs=pl.BlockSpec((BATCH, OUT_FEATURES), lambda k: (0, 0)),
        out_shape=jax.ShapeDtypeStruct((BATCH, OUT_FEATURES), x.dtype),
        scratch_shapes=[
            pltpu.VMEM((BATCH, IN_FEATURES), jnp.float32),
            pltpu.VMEM((BATCH, COMPRESSED), jnp.float32),
            pltpu.VMEM((COMPRESSED, IN_FEATURES), jnp.float32),
            pltpu.VMEM((OUT_FEATURES, COMPRESSED), jnp.float32),
            pltpu.SemaphoreType.DMA,
            pltpu.SemaphoreType.DMA,
        ],
    )(x, sw_row, we, compressor_w, compressed_net_w)
